# baseline (speedup 1.0000x reference)
import functools

import jax
import jax.numpy as jnp
from jax import lax
from jax.experimental import pallas as pl
from jax.experimental.pallas import tpu as pltpu
from jax.experimental.pallas import tpu_sc as plsc

VOCAB = 1000001
EMBED_DIM = 64
BATCH = 4096
SEQ_LEN = 50
VPAD = 1000008
NBLK = VOCAB // 128
TAIL = VOCAB - NBLK * 128
NC = 2
NS = 16
NW = NC * NS
BLK_W = -(-NBLK // NW)
BROWS_W = BATCH // NW

_mesh = plsc.VectorSubcoreMesh(core_axis_name="c", subcore_axis_name="s")
_params = pltpu.CompilerParams(use_tc_tiling_on_sc=True,
                               needs_layout_passes=False)


@functools.partial(
    pl.kernel,
    mesh=_mesh,
    out_type=jax.ShapeDtypeStruct((VPAD, 128), jnp.float32),
    scratch_types=[
        pltpu.VMEM((EMBED_DIM, 128), jnp.float32),
        pltpu.VMEM((EMBED_DIM, 128), jnp.float32),
        pltpu.VMEM((128, 128), jnp.float32),
        pltpu.VMEM((128, 128), jnp.float32),
        pltpu.SemaphoreType.DMA,
        pltpu.SemaphoreType.DMA,
        pltpu.SemaphoreType.DMA,
        pltpu.SemaphoreType.DMA,
    ],
    compiler_params=_params,
)
def _sc_transpose(tT_hbm, tail_hbm, out_hbm, src0, src1, dst0, dst1,
                  is0, is1, os0, os1):
    wid = lax.axis_index("s") * NC + lax.axis_index("c")
    srcs = (src0, src1)
    dsts = (dst0, dst1)
    isems = (is0, is1)
    osems = (os0, os1)
    iota = lax.iota(jnp.int32, 16)
    rows = [ib * 16 + iota for ib in range(8)]

    def blk_of(step):
        return step * NW + wid

    def fire_in(blk, k):
        pltpu.async_copy(
            tT_hbm.at[:, pl.ds(pl.multiple_of(blk * 128, 128), 128)],
            srcs[k], isems[k])

    def transpose_block(src, dst):
        for d in range(EMBED_DIM):
            col = jnp.full((16,), d, jnp.int32)
            for ib in range(8):
                v = src[d, pl.ds(ib * 16, 16)]
                plsc.store_scatter(dst, [rows[ib], col], v)

    def do_step(step, k, nk, last):
        blk = blk_of(step)

        @pl.when(blk < NBLK)
        def _():
            if not last:
                @pl.when(blk_of(step + 1) < NBLK)
                def _():
                    fire_in(blk_of(step + 1), nk)
            pltpu.make_async_copy(
                tT_hbm.at[:, pl.ds(0, 128)], srcs[k], isems[k]).wait()

            @pl.when((step >= 2) & (blk_of(step - 2) < NBLK))
            def _():
                pltpu.make_async_copy(
                    dsts[k], out_hbm.at[pl.ds(0, 128)], osems[k]).wait()
            transpose_block(srcs[k], dsts[k])
            pltpu.async_copy(
                dsts[k], out_hbm.at[pl.ds(pl.multiple_of(blk * 128, 8), 128)],
                osems[k])

    @pl.when(blk_of(0) < NBLK)
    def _():
        fire_in(blk_of(0), 0)

    def pair(g, carry):
        do_step(g * 2, 0, 1, False)
        do_step(g * 2 + 1, 1, 0, False)
        return carry
    lax.fori_loop(0, (BLK_W - 1) // 2, pair, 0)
    do_step(BLK_W - 1, (BLK_W - 1) & 1, BLK_W & 1, True)

    @pl.when(blk_of(BLK_W - 1) < NBLK)
    def _():
        pltpu.make_async_copy(
            dsts[(BLK_W - 1) & 1], out_hbm.at[pl.ds(0, 128)],
            osems[(BLK_W - 1) & 1]).wait()
    @pl.when(blk_of(BLK_W - 2) < NBLK)
    def _():
        pltpu.make_async_copy(
            dsts[(BLK_W - 2) & 1], out_hbm.at[pl.ds(0, 128)],
            osems[(BLK_W - 2) & 1]).wait()

    @pl.when(wid == 0)
    def _():
        pltpu.sync_copy(tail_hbm, dst0.at[pl.ds(0, TAIL + 7)])
        pltpu.sync_copy(dst0.at[pl.ds(0, TAIL + 7)],
                        out_hbm.at[pl.ds(NBLK * 128, TAIL + 7)])


@functools.partial(
    pl.kernel,
    mesh=_mesh,
    out_type=jax.ShapeDtypeStruct((BATCH, SEQ_LEN, 128), jnp.float32),
    scratch_types=[
        pltpu.VMEM((SEQ_LEN, BROWS_W), jnp.int32),
        pltpu.VMEM((BROWS_W, SEQ_LEN), jnp.int32),
        pltpu.VMEM((SEQ_LEN, 128), jnp.float32),
        pltpu.VMEM((SEQ_LEN, 128), jnp.float32),
        pltpu.SemaphoreType.DMA,
        pltpu.SemaphoreType.DMA,
        pltpu.SemaphoreType.DMA,
        pltpu.SemaphoreType.DMA,
    ],
    compiler_params=_params,
)
def _sc_gather(idsT_hbm, tab_hbm, out_hbm, idx_v, idxT_v, buf0, buf1,
               gs0, gs1, os0, os1):
    wid = lax.axis_index("s") * NC + lax.axis_index("c")
    b0 = wid * BROWS_W
    bufs = (buf0, buf1)
    gsems = (gs0, gs1)
    osems = (os0, os1)
    iota = lax.iota(jnp.int32, 16)
    rows = [ib * 16 + iota for ib in range(8)]

    pltpu.sync_copy(
        idsT_hbm.at[:, pl.ds(pl.multiple_of(b0, 128), BROWS_W)], idx_v)
    for s in range(SEQ_LEN):
        col = jnp.full((16,), s, jnp.int32)
        for ib in range(8):
            v = idx_v[s, pl.ds(ib * 16, 16)]
            plsc.store_scatter(idxT_v, [rows[ib], col], v)

    def fire(jj, k):
        pltpu.async_copy(tab_hbm.at[idxT_v.at[jj]], bufs[k], gsems[k])

    def drain_gather(jj, k):
        pltpu.make_async_copy(
            tab_hbm.at[idxT_v.at[jj]], bufs[k], gsems[k]).wait()

    def wait_out(k):
        pltpu.make_async_copy(
            bufs[k], out_hbm.at[0], osems[k]).wait()

    fire(0, 0)

    def step(g, carry):
        for kk in range(2):
            jj = g * 2 + kk
            k = kk
            nk = 1 - kk

            @pl.when(jj + 1 < BROWS_W)
            def _():
                fire(jj + 1, nk)
            drain_gather(jj, k)
            @pl.when(jj >= 2)
            def _():
                wait_out(k)
            pltpu.async_copy(bufs[k], out_hbm.at[b0 + jj], osems[k])
        return carry
    lax.fori_loop(0, BROWS_W // 2, step, 0)
    wait_out(0)
    wait_out(1)


def kernel(input_ids, table):
    tT = table.T
    tail = jnp.pad(lax.slice(table, (NBLK * 128, 0), (VOCAB, EMBED_DIM)),
                   ((0, 7), (0, 128 - EMBED_DIM)))
    scratch = _sc_transpose(tT, tail)
    idsT = input_ids.T.astype(jnp.int32)
    out = _sc_gather(idsT, scratch)
    return out[:, :, :EMBED_DIM]


# speedup vs baseline: 1.2724x; 1.2724x over previous
import functools

import jax
import jax.numpy as jnp
from jax import lax
from jax.experimental import pallas as pl
from jax.experimental.pallas import tpu as pltpu
from jax.experimental.pallas import tpu_sc as plsc

VOCAB = 1000001
EMBED_DIM = 64
BATCH = 4096
SEQ_LEN = 50
VPAD = 1000008
NBLK = VOCAB // 128
TAIL = VOCAB - NBLK * 128
NC = 2
NS = 16
NW = NC * NS
BLK_W = -(-NBLK // NW)
BROWS_W = BATCH // NW

_mesh = plsc.VectorSubcoreMesh(core_axis_name="c", subcore_axis_name="s")
_params = pltpu.CompilerParams(use_tc_tiling_on_sc=True,
                               needs_layout_passes=False)


@functools.partial(
    pl.kernel,
    mesh=_mesh,
    out_type=jax.ShapeDtypeStruct((VPAD, 128), jnp.float32),
    scratch_types=[
        pltpu.VMEM((EMBED_DIM, 128), jnp.float32),
        pltpu.VMEM((EMBED_DIM, 128), jnp.float32),
        pltpu.VMEM((128, 128), jnp.float32),
        pltpu.VMEM((128, 128), jnp.float32),
        pltpu.SemaphoreType.DMA,
        pltpu.SemaphoreType.DMA,
        pltpu.SemaphoreType.DMA,
        pltpu.SemaphoreType.DMA,
    ],
    compiler_params=_params,
)
def _sc_transpose(tT_hbm, tail_hbm, out_hbm, src0, src1, dst0, dst1,
                  is0, is1, os0, os1):
    wid = lax.axis_index("s") * NC + lax.axis_index("c")
    srcs = (src0, src1)
    dsts = (dst0, dst1)
    isems = (is0, is1)
    osems = (os0, os1)
    iota = lax.iota(jnp.int32, 16)
    rows = [ib * 16 + iota for ib in range(8)]

    def blk_of(step):
        return step * NW + wid

    def fire_in(blk, k):
        pltpu.async_copy(
            tT_hbm.at[:, pl.ds(pl.multiple_of(blk * 128, 128), 128)],
            srcs[k], isems[k])

    cols = [jnp.full((16,), d, jnp.int32) for d in range(EMBED_DIM)]

    def transpose_block(src, dst):
        for ib in range(8):
            row = rows[ib]
            for dg in range(8):
                vs = [src[dg * 8 + dd, pl.ds(ib * 16, 16)]
                      for dd in range(8)]
                for dd in range(8):
                    plsc.store_scatter(dst, [row, cols[dg * 8 + dd]], vs[dd])

    def do_step(step, k, nk, last):
        blk = blk_of(step)

        @pl.when(blk < NBLK)
        def _():
            if not last:
                @pl.when(blk_of(step + 1) < NBLK)
                def _():
                    fire_in(blk_of(step + 1), nk)
            pltpu.make_async_copy(
                tT_hbm.at[:, pl.ds(0, 128)], srcs[k], isems[k]).wait()

            @pl.when((step >= 2) & (blk_of(step - 2) < NBLK))
            def _():
                pltpu.make_async_copy(
                    dsts[k], out_hbm.at[pl.ds(0, 128)], osems[k]).wait()
            transpose_block(srcs[k], dsts[k])
            pltpu.async_copy(
                dsts[k], out_hbm.at[pl.ds(pl.multiple_of(blk * 128, 8), 128)],
                osems[k])

    @pl.when(blk_of(0) < NBLK)
    def _():
        fire_in(blk_of(0), 0)

    def pair(g, carry):
        do_step(g * 2, 0, 1, False)
        do_step(g * 2 + 1, 1, 0, False)
        return carry
    lax.fori_loop(0, (BLK_W - 1) // 2, pair, 0)
    do_step(BLK_W - 1, (BLK_W - 1) & 1, BLK_W & 1, True)

    @pl.when(blk_of(BLK_W - 1) < NBLK)
    def _():
        pltpu.make_async_copy(
            dsts[(BLK_W - 1) & 1], out_hbm.at[pl.ds(0, 128)],
            osems[(BLK_W - 1) & 1]).wait()
    @pl.when(blk_of(BLK_W - 2) < NBLK)
    def _():
        pltpu.make_async_copy(
            dsts[(BLK_W - 2) & 1], out_hbm.at[pl.ds(0, 128)],
            osems[(BLK_W - 2) & 1]).wait()

    @pl.when(wid == 0)
    def _():
        pltpu.sync_copy(tail_hbm, dst0.at[pl.ds(0, TAIL + 7)])
        pltpu.sync_copy(dst0.at[pl.ds(0, TAIL + 7)],
                        out_hbm.at[pl.ds(NBLK * 128, TAIL + 7)])


@functools.partial(
    pl.kernel,
    mesh=_mesh,
    out_type=jax.ShapeDtypeStruct((BATCH, SEQ_LEN, 128), jnp.float32),
    scratch_types=[
        pltpu.VMEM((SEQ_LEN, BROWS_W), jnp.int32),
        pltpu.VMEM((BROWS_W, SEQ_LEN), jnp.int32),
        pltpu.VMEM((SEQ_LEN, 128), jnp.float32),
        pltpu.VMEM((SEQ_LEN, 128), jnp.float32),
        pltpu.SemaphoreType.DMA,
        pltpu.SemaphoreType.DMA,
        pltpu.SemaphoreType.DMA,
        pltpu.SemaphoreType.DMA,
    ],
    compiler_params=_params,
)
def _sc_gather(idsT_hbm, tab_hbm, out_hbm, idx_v, idxT_v, buf0, buf1,
               gs0, gs1, os0, os1):
    wid = lax.axis_index("s") * NC + lax.axis_index("c")
    b0 = wid * BROWS_W
    bufs = (buf0, buf1)
    gsems = (gs0, gs1)
    osems = (os0, os1)
    iota = lax.iota(jnp.int32, 16)
    rows = [ib * 16 + iota for ib in range(8)]

    pltpu.sync_copy(
        idsT_hbm.at[:, pl.ds(pl.multiple_of(b0, 128), BROWS_W)], idx_v)
    for s in range(SEQ_LEN):
        col = jnp.full((16,), s, jnp.int32)
        for ib in range(8):
            v = idx_v[s, pl.ds(ib * 16, 16)]
            plsc.store_scatter(idxT_v, [rows[ib], col], v)

    def fire(jj, k):
        pltpu.async_copy(tab_hbm.at[idxT_v.at[jj]], bufs[k], gsems[k])

    def drain_gather(jj, k):
        pltpu.make_async_copy(
            tab_hbm.at[idxT_v.at[jj]], bufs[k], gsems[k]).wait()

    def wait_out(k):
        pltpu.make_async_copy(
            bufs[k], out_hbm.at[0], osems[k]).wait()

    fire(0, 0)

    def step(g, carry):
        for kk in range(2):
            jj = g * 2 + kk
            k = kk
            nk = 1 - kk

            @pl.when(jj + 1 < BROWS_W)
            def _():
                fire(jj + 1, nk)
            drain_gather(jj, k)
            @pl.when(jj >= 2)
            def _():
                wait_out(k)
            pltpu.async_copy(bufs[k], out_hbm.at[b0 + jj], osems[k])
        return carry
    lax.fori_loop(0, BROWS_W // 2, step, 0)
    wait_out(0)
    wait_out(1)


def kernel(input_ids, table):
    tT = table.T
    tail = jnp.pad(lax.slice(table, (NBLK * 128, 0), (VOCAB, EMBED_DIM)),
                   ((0, 7), (0, 128 - EMBED_DIM)))
    scratch = _sc_transpose(tT, tail)
    idsT = input_ids.T.astype(jnp.int32)
    out = _sc_gather(idsT, scratch)
    return out[:, :, :EMBED_DIM]


# speedup vs baseline: 1.3523x; 1.0627x over previous
import functools

import jax
import jax.numpy as jnp
from jax import lax
from jax.experimental import pallas as pl
from jax.experimental.pallas import tpu as pltpu
from jax.experimental.pallas import tpu_sc as plsc

VOCAB = 1000001
EMBED_DIM = 64
BATCH = 4096
SEQ_LEN = 50
VPAD = 1000008
NBLK = VOCAB // 128
TAIL = VOCAB - NBLK * 128
NC = 2
NS = 16
NW = NC * NS
BLK_W = -(-NBLK // NW)
BROWS_W = BATCH // NW

_mesh = plsc.VectorSubcoreMesh(core_axis_name="c", subcore_axis_name="s")
_params = pltpu.CompilerParams(use_tc_tiling_on_sc=True,
                               needs_layout_passes=False)


@functools.partial(
    pl.kernel,
    mesh=_mesh,
    out_type=jax.ShapeDtypeStruct((VPAD, 128), jnp.float32),
    scratch_types=[
        pltpu.VMEM((EMBED_DIM, 128), jnp.float32),
        pltpu.VMEM((EMBED_DIM, 128), jnp.float32),
        pltpu.VMEM((128, 128), jnp.float32),
        pltpu.VMEM((128, 128), jnp.float32),
        pltpu.SemaphoreType.DMA,
        pltpu.SemaphoreType.DMA,
        pltpu.SemaphoreType.DMA,
        pltpu.SemaphoreType.DMA,
    ],
    compiler_params=_params,
)
def _sc_transpose(tT_hbm, tail_hbm, out_hbm, src0, src1, dst0, dst1,
                  is0, is1, os0, os1):
    wid = lax.axis_index("s") * NC + lax.axis_index("c")
    srcs = (src0, src1)
    dsts = (dst0, dst1)
    isems = (is0, is1)
    osems = (os0, os1)
    iota = lax.iota(jnp.int32, 16)
    rows = [ib * 16 + iota for ib in range(8)]

    def blk_of(step):
        return step * NW + wid

    def fire_in(blk, k):
        pltpu.async_copy(
            tT_hbm.at[:, pl.ds(pl.multiple_of(blk * 128, 128), 128)],
            srcs[k], isems[k])

    iota16 = lax.iota(jnp.int32, 16)

    def transpose_block(src, dst):
        @plsc.parallel_loop(0, EMBED_DIM * 8, step=1, unroll=8)
        def _(t):
            d = t % EMBED_DIM
            ib = t // EMBED_DIM
            col = jnp.full((16,), 0, jnp.int32) + d
            row = ib * 16 + iota16
            v = src[d, pl.ds(pl.multiple_of(ib * 16, 16), 16)]
            plsc.store_scatter(dst, [row, col], v)

    def do_step(step, k, nk, last):
        blk = blk_of(step)

        @pl.when(blk < NBLK)
        def _():
            if not last:
                @pl.when(blk_of(step + 1) < NBLK)
                def _():
                    fire_in(blk_of(step + 1), nk)
            pltpu.make_async_copy(
                tT_hbm.at[:, pl.ds(0, 128)], srcs[k], isems[k]).wait()

            @pl.when((step >= 2) & (blk_of(step - 2) < NBLK))
            def _():
                pltpu.make_async_copy(
                    dsts[k], out_hbm.at[pl.ds(0, 128)], osems[k]).wait()
            transpose_block(srcs[k], dsts[k])
            pltpu.async_copy(
                dsts[k], out_hbm.at[pl.ds(pl.multiple_of(blk * 128, 8), 128)],
                osems[k])

    @pl.when(blk_of(0) < NBLK)
    def _():
        fire_in(blk_of(0), 0)

    def pair(g, carry):
        do_step(g * 2, 0, 1, False)
        do_step(g * 2 + 1, 1, 0, False)
        return carry
    lax.fori_loop(0, (BLK_W - 1) // 2, pair, 0)
    do_step(BLK_W - 1, (BLK_W - 1) & 1, BLK_W & 1, True)

    @pl.when(blk_of(BLK_W - 1) < NBLK)
    def _():
        pltpu.make_async_copy(
            dsts[(BLK_W - 1) & 1], out_hbm.at[pl.ds(0, 128)],
            osems[(BLK_W - 1) & 1]).wait()
    @pl.when(blk_of(BLK_W - 2) < NBLK)
    def _():
        pltpu.make_async_copy(
            dsts[(BLK_W - 2) & 1], out_hbm.at[pl.ds(0, 128)],
            osems[(BLK_W - 2) & 1]).wait()

    @pl.when(wid == 0)
    def _():
        pltpu.sync_copy(tail_hbm, dst0.at[pl.ds(0, TAIL + 7)])
        pltpu.sync_copy(dst0.at[pl.ds(0, TAIL + 7)],
                        out_hbm.at[pl.ds(NBLK * 128, TAIL + 7)])


@functools.partial(
    pl.kernel,
    mesh=_mesh,
    out_type=jax.ShapeDtypeStruct((BATCH, SEQ_LEN, 128), jnp.float32),
    scratch_types=[
        pltpu.VMEM((SEQ_LEN, BROWS_W), jnp.int32),
        pltpu.VMEM((BROWS_W, SEQ_LEN), jnp.int32),
        pltpu.VMEM((SEQ_LEN, 128), jnp.float32),
        pltpu.VMEM((SEQ_LEN, 128), jnp.float32),
        pltpu.SemaphoreType.DMA,
        pltpu.SemaphoreType.DMA,
        pltpu.SemaphoreType.DMA,
        pltpu.SemaphoreType.DMA,
    ],
    compiler_params=_params,
)
def _sc_gather(idsT_hbm, tab_hbm, out_hbm, idx_v, idxT_v, buf0, buf1,
               gs0, gs1, os0, os1):
    wid = lax.axis_index("s") * NC + lax.axis_index("c")
    b0 = wid * BROWS_W
    bufs = (buf0, buf1)
    gsems = (gs0, gs1)
    osems = (os0, os1)
    iota = lax.iota(jnp.int32, 16)
    rows = [ib * 16 + iota for ib in range(8)]

    pltpu.sync_copy(
        idsT_hbm.at[:, pl.ds(pl.multiple_of(b0, 128), BROWS_W)], idx_v)
    for s in range(SEQ_LEN):
        col = jnp.full((16,), s, jnp.int32)
        for ib in range(8):
            v = idx_v[s, pl.ds(ib * 16, 16)]
            plsc.store_scatter(idxT_v, [rows[ib], col], v)

    def fire(jj, k):
        pltpu.async_copy(tab_hbm.at[idxT_v.at[jj]], bufs[k], gsems[k])

    def drain_gather(jj, k):
        pltpu.make_async_copy(
            tab_hbm.at[idxT_v.at[jj]], bufs[k], gsems[k]).wait()

    def wait_out(k):
        pltpu.make_async_copy(
            bufs[k], out_hbm.at[0], osems[k]).wait()

    fire(0, 0)

    def step(g, carry):
        for kk in range(2):
            jj = g * 2 + kk
            k = kk
            nk = 1 - kk

            @pl.when(jj + 1 < BROWS_W)
            def _():
                fire(jj + 1, nk)
            drain_gather(jj, k)
            @pl.when(jj >= 2)
            def _():
                wait_out(k)
            pltpu.async_copy(bufs[k], out_hbm.at[b0 + jj], osems[k])
        return carry
    lax.fori_loop(0, BROWS_W // 2, step, 0)
    wait_out(0)
    wait_out(1)


def kernel(input_ids, table):
    tT = table.T
    tail = jnp.pad(lax.slice(table, (NBLK * 128, 0), (VOCAB, EMBED_DIM)),
                   ((0, 7), (0, 128 - EMBED_DIM)))
    scratch = _sc_transpose(tT, tail)
    idsT = input_ids.T.astype(jnp.int32)
    out = _sc_gather(idsT, scratch)
    return out[:, :, :EMBED_DIM]
